# baseline (speedup 1.0000x reference)
import functools

import jax
import jax.numpy as jnp
from jax import lax
from jax.experimental import pallas as pl
from jax.experimental.pallas import tpu as pltpu
from jax.experimental.pallas import tpu_sc as plsc

N = 10000
E = 320000
D = 128

NC = 2
NS = 16
NW = NC * NS

CH = 128

NPT = 632
NPT_LAST = N - 15 * NPT

_sc_mesh = plsc.VectorSubcoreMesh(core_axis_name="c", subcore_axis_name="s")


def _make_sc_gather_add(ne):
    orows = ne // NW
    cf = orows // CH
    ct = orows - cf * CH
    assert ne % NW == 0 and ct % 8 == 0 and ct > 0 and cf % 2 == 1

    @functools.partial(
        pl.kernel,
        out_type=jax.ShapeDtypeStruct((ne, D), jnp.float32),
        mesh=_sc_mesh,
        scratch_types=[
            pltpu.VMEM((CH,), jnp.int32),
            pltpu.VMEM((CH,), jnp.int32),
            pltpu.VMEM((CH,), jnp.int32),
            pltpu.VMEM((CH,), jnp.int32),
            pltpu.VMEM((ct,), jnp.int32),
            pltpu.VMEM((ct,), jnp.int32),
            pltpu.VMEM((CH, D), jnp.float32),
            pltpu.VMEM((CH, D), jnp.float32),
            pltpu.VMEM((CH, D), jnp.float32),
            pltpu.VMEM((CH, D), jnp.float32),
            pltpu.VMEM((ct, D), jnp.float32),
            pltpu.VMEM((ct, D), jnp.float32),
            pltpu.SemaphoreType.DMA,
            pltpu.SemaphoreType.DMA,
            pltpu.SemaphoreType.DMA,
            pltpu.SemaphoreType.DMA,
            pltpu.SemaphoreType.DMA,
            pltpu.SemaphoreType.DMA,
            pltpu.SemaphoreType.DMA,
            pltpu.SemaphoreType.DMA,
            pltpu.SemaphoreType.DMA,
            pltpu.SemaphoreType.DMA,
        ],
    )
    def sc_gather_add(uv_hbm, idx_hbm, out_hbm,
                      ir0, ir1, is0, is1, irt, ist,
                      ra0, ra1, rb0, rb1, rat, rbt,
                      sir0, sir1, sis0, sis1, sga0, sga1, sgb0, sgb1, ss0, ss1):
        c = lax.axis_index("c")
        s = lax.axis_index("s")
        base_w = (s * NC + c) * orows
        ir, isv = (ir0, ir1), (is0, is1)
        ra, rb = (ra0, ra1), (rb0, rb1)
        sir, sis = (sir0, sir1), (sis0, sis1)
        sga, sgb = (sga0, sga1), (sgb0, sgb1)
        ss = (ss0, ss1)

        def start_idx(j, b):
            pltpu.async_copy(idx_hbm.at[pl.ds(base_w + j * CH, CH)], ir[b], sir[b])
            pltpu.async_copy(idx_hbm.at[pl.ds(ne + base_w + j * CH, CH)],
                             isv[b], sis[b])

        def wait_idx(b):
            pltpu.make_async_copy(idx_hbm.at[pl.ds(base_w, CH)], ir[b], sir[b]).wait()
            pltpu.make_async_copy(idx_hbm.at[pl.ds(base_w, CH)], isv[b], sis[b]).wait()

        def start_gathers(b):
            pltpu.async_copy(uv_hbm.at[ir[b]], ra[b], sga[b])
            pltpu.async_copy(uv_hbm.at[isv[b]], rb[b], sgb[b])

        def wait_gathers(b):
            pltpu.make_async_copy(uv_hbm.at[ir[b]], ra[b], sga[b]).wait()
            pltpu.make_async_copy(uv_hbm.at[isv[b]], rb[b], sgb[b]).wait()

        def vadd(dst, src, nrow):
            @pl.loop(0, nrow)
            def _(r):
                for q in range(D // 16):
                    sl = pl.ds(q * 16, 16)
                    dst[r, sl] = dst[r, sl] + src[r, sl]

        def start_store(j, b):
            pltpu.async_copy(ra[b], out_hbm.at[pl.ds(base_w + j * CH, CH)], ss[b])

        def wait_store(b):
            pltpu.make_async_copy(ra[b], out_hbm.at[pl.ds(base_w, CH)], ss[b]).wait()

        def chunk(j, b, wait_prev_store, start_next):
            nb = 1 - b
            if start_next:
                start_idx(j + 1, nb)
            wait_gathers(b)
            if start_next:
                wait_idx(nb)
                if wait_prev_store:
                    wait_store(nb)
                start_gathers(nb)
            vadd(ra[b], rb[b], CH)
            start_store(j, b)

        pltpu.sync_copy(idx_hbm.at[pl.ds(base_w, CH)], ir0)
        pltpu.sync_copy(idx_hbm.at[pl.ds(ne + base_w, CH)], is0)
        start_gathers(0)
        chunk(0, 0, wait_prev_store=False, start_next=True)

        @pl.loop(1, cf - 2, step=2)
        def _(j0):
            chunk(j0, 1, wait_prev_store=True, start_next=True)
            chunk(j0 + 1, 0, wait_prev_store=True, start_next=True)

        chunk(cf - 2, 1, wait_prev_store=True, start_next=True)
        chunk(cf - 1, 0, wait_prev_store=False, start_next=False)

        tb = base_w + cf * CH
        pltpu.sync_copy(idx_hbm.at[pl.ds(tb, ct)], irt)
        pltpu.sync_copy(idx_hbm.at[pl.ds(ne + tb, ct)], ist)
        pltpu.async_copy(uv_hbm.at[irt], rat, sga0).wait()
        pltpu.async_copy(uv_hbm.at[ist], rbt, sgb0).wait()
        vadd(rat, rbt, ct)
        pltpu.sync_copy(rat, out_hbm.at[pl.ds(tb, ct)])

        wait_store(1)
        wait_store(0)

    return sc_gather_add


def _make_sc_scatter(ne):
    srows = ne // NS
    sfull = srows // CH
    stail = srows - sfull * CH
    assert ne % NS == 0 and stail % 8 == 0 and stail > 0 and sfull % 2 == 0

    @functools.partial(
        pl.kernel,
        out_type=jax.ShapeDtypeStruct((2 * N, D), jnp.float32),
        mesh=_sc_mesh,
        scratch_types=[
            pltpu.VMEM((CH,), jnp.int32),
            pltpu.VMEM((CH,), jnp.int32),
            pltpu.VMEM((stail,), jnp.int32),
            pltpu.VMEM((CH, D), jnp.float32),
            pltpu.VMEM((CH, D), jnp.float32),
            pltpu.VMEM((stail, D), jnp.float32),
            pltpu.VMEM_SHARED((N, D), jnp.float32),
            pltpu.SemaphoreType.DMA,
            pltpu.SemaphoreType.DMA,
            pltpu.SemaphoreType.DMA,
            pltpu.SemaphoreType.DMA,
            pltpu.SemaphoreType.DMA,
            pltpu.SemaphoreType.DMA,
        ],
    )
    def sc_scatter(msg_hbm, idx_hbm, zero_hbm, out_hbm,
                   idx0, idx1, idxt, r0, r1, rt, acc,
                   si0, si1, sm0, sm1, sa0, sa1):
        c = lax.axis_index("c")
        s = lax.axis_index("s")
        idxb, rows = (idx0, idx1), (r0, r1)
        si, sm, sa = (si0, si1), (sm0, sm1), (sa0, sa1)
        ebase = s * srows

        def start_loads(j, b):
            pltpu.async_copy(idx_hbm.at[pl.ds(c * ne + ebase + j * CH, CH)],
                             idxb[b], si[b])
            pltpu.async_copy(msg_hbm.at[pl.ds(ebase + j * CH, CH)], rows[b], sm[b])

        def wait_loads(b):
            pltpu.make_async_copy(idx_hbm.at[pl.ds(ebase, CH)], idxb[b], si[b]).wait()
            pltpu.make_async_copy(msg_hbm.at[pl.ds(ebase, CH)], rows[b], sm[b]).wait()

        def start_scatter(b):
            pltpu.async_copy(rows[b], acc.at[idxb[b]], sa[b], add=True)

        def wait_scatter(b):
            pltpu.make_async_copy(rows[b], acc.at[idxb[b]], sa[b]).wait()

        start_loads(0, 0)

        abase = s * NPT
        pltpu.sync_copy(zero_hbm.at[pl.ds(0, NPT_LAST)],
                        acc.at[pl.ds(abase, NPT_LAST)])

        @pl.when(s < NS - 1)
        def _():
            pltpu.sync_copy(zero_hbm.at[pl.ds(0, NPT - NPT_LAST)],
                            acc.at[pl.ds(abase + NPT_LAST, NPT - NPT_LAST)])

        plsc.subcore_barrier()

        def chunk(j, b, wait_prev_scatter, start_next):
            nb = 1 - b
            if wait_prev_scatter:
                wait_scatter(nb)
            if start_next:
                start_loads(j + 1, nb)
            wait_loads(b)
            start_scatter(b)

        chunk(0, 0, wait_prev_scatter=False, start_next=True)

        @pl.loop(1, sfull - 1, step=2)
        def _(j0):
            chunk(j0, 1, wait_prev_scatter=True, start_next=True)
            chunk(j0 + 1, 0, wait_prev_scatter=True, start_next=True)

        chunk(sfull - 1, 1, wait_prev_scatter=True, start_next=False)
        wait_scatter(1)

        tb = ebase + sfull * CH
        pltpu.sync_copy(idx_hbm.at[pl.ds(c * ne + tb, stail)], idxt)
        pltpu.sync_copy(msg_hbm.at[pl.ds(tb, stail)], rt)
        pltpu.sync_copy(rt, acc.at[idxt], add=True)
        plsc.subcore_barrier()

        pltpu.sync_copy(acc.at[pl.ds(abase, NPT_LAST)],
                        out_hbm.at[pl.ds(c * N + abase, NPT_LAST)])

        @pl.when(s < NS - 1)
        def _():
            pltpu.sync_copy(acc.at[pl.ds(abase + NPT_LAST, NPT - NPT_LAST)],
                            out_hbm.at[pl.ds(c * N + abase + NPT_LAST,
                                             NPT - NPT_LAST)])

    return sc_scatter


def _uv_body(x_ref, w_ref, out_ref):
    out_ref[...] = jnp.dot(x_ref[...], w_ref[...],
                           preferred_element_type=jnp.float32)


def _uv_precompute(x, We1, block=1000):
    nb = N // block
    return pl.pallas_call(
        _uv_body,
        grid=(2, nb),
        in_specs=[
            pl.BlockSpec((block, D), lambda k, i: (i, 0)),
            pl.BlockSpec((D, D), lambda k, i: (k + 1, 0)),
        ],
        out_specs=pl.BlockSpec((block, D), lambda k, i: (k * nb + i, 0)),
        out_shape=jax.ShapeDtypeStruct((2 * N, D), jnp.float32),
    )(x, We1)


def _edge_body(ea_ref, gs_ref, w1_ref, b1_ref, w2_ref, b2_ref,
               w3_ref, b3_ref, g_ref, b_ref, msg_ref, eout_ref):
    ea = ea_ref[...]
    f32 = jnp.float32
    h = jnp.dot(ea, w1_ref[0:D, :], preferred_element_type=f32)
    h += gs_ref[...]
    h = jnp.tanh(h + b1_ref[...])
    h = jnp.tanh(jnp.dot(h, w2_ref[...], preferred_element_type=f32) + b2_ref[...])
    h = jnp.dot(h, w3_ref[...], preferred_element_type=f32) + b3_ref[...]
    mu = jnp.mean(h, axis=-1, keepdims=True)
    hc = h - mu
    var = jnp.mean(hc * hc, axis=-1, keepdims=True)
    m = hc * lax.rsqrt(var + 1e-5) * g_ref[...] + b_ref[...]
    msg_ref[...] = m
    eout_ref[...] = ea + m


def _edge_mlp(edge_attr, gs, We1, be1, We2, be2, We3, be3, g, b, ne, block,
              ea_off, eout_prev=None):
    nb = ne // block
    row = lambda i: (i, 0)
    eout_row = lambda i: (i + ea_off, 0)
    full = lambda shape: pl.BlockSpec(shape, lambda i: (0, 0))
    in_specs = [
        pl.BlockSpec((block, D), eout_row),
        pl.BlockSpec((block, D), row),
        full((3 * D, D)), full((1, D)),
        full((D, D)), full((1, D)),
        full((D, D)), full((1, D)),
        full((1, D)), full((1, D)),
    ]
    args = [edge_attr, gs, We1, be1, We2, be2, We3, be3, g, b]
    aliases = {}
    body = _edge_body
    if eout_prev is not None:
        in_specs.append(pl.BlockSpec((8, D), lambda i: (0, 0)))
        args.append(eout_prev)
        aliases = {len(args) - 1: 1}
        body = lambda *refs: _edge_body(*refs[:10], *refs[11:])
    return pl.pallas_call(
        body,
        grid=(nb,),
        in_specs=in_specs,
        out_specs=[pl.BlockSpec((block, D), row),
                   pl.BlockSpec((block, D), eout_row)],
        out_shape=[
            jax.ShapeDtypeStruct((ne, D), jnp.float32),
            jax.ShapeDtypeStruct((E, D), jnp.float32),
        ],
        input_output_aliases=aliases,
    )(*args)


def _node_body(x_ref, ar_ref, as_ref, br_ref, bs_ref, w1_ref, b1_ref,
               w2_ref, b2_ref, w3_ref, b3_ref, g_ref, b_ref, out_ref):
    x = x_ref[...]
    agg = (ar_ref[...] + br_ref[...]) - (as_ref[...] + bs_ref[...])
    f32 = jnp.float32
    h = jnp.dot(x, w1_ref[0:D, :], preferred_element_type=f32)
    h += jnp.dot(agg, w1_ref[D:2 * D, :], preferred_element_type=f32)
    h = jnp.tanh(h + b1_ref[...])
    h = jnp.tanh(jnp.dot(h, w2_ref[...], preferred_element_type=f32) + b2_ref[...])
    h = jnp.dot(h, w3_ref[...], preferred_element_type=f32) + b3_ref[...]
    mu = jnp.mean(h, axis=-1, keepdims=True)
    hc = h - mu
    var = jnp.mean(hc * hc, axis=-1, keepdims=True)
    out_ref[...] = hc * lax.rsqrt(var + 1e-5) * g_ref[...] + b_ref[...] + x


def _node_mlp(x, aggA, aggB, Wn1, bn1, Wn2, bn2, Wn3, bn3, g, b, block):
    nb = N // block
    row = lambda i: (i, 0)
    shift = lambda i: (i + nb, 0)
    full = lambda shape: pl.BlockSpec(shape, lambda i: (0, 0))
    return pl.pallas_call(
        _node_body,
        grid=(nb,),
        in_specs=[
            pl.BlockSpec((block, D), row),
            pl.BlockSpec((block, D), row),
            pl.BlockSpec((block, D), shift),
            pl.BlockSpec((block, D), row),
            pl.BlockSpec((block, D), shift),
            full((2 * D, D)), full((1, D)),
            full((D, D)), full((1, D)),
            full((D, D)), full((1, D)),
            full((1, D)), full((1, D)),
        ],
        out_specs=pl.BlockSpec((block, D), row),
        out_shape=jax.ShapeDtypeStruct((N, D), jnp.float32),
    )(x, aggA, aggA, aggB, aggB, Wn1, bn1, Wn2, bn2, Wn3, bn3, g, b)


EH = E // 2
_sc_gather_add_h = _make_sc_gather_add(EH)
_sc_scatter_h = _make_sc_scatter(EH)


@jax.jit
def kernel(x, senders, receivers, edge_attr,
           We1, be1, We2, be2, We3, be3, lne_g, lne_b,
           Wn1, bn1, Wn2, bn2, Wn3, bn3, lnn_g, lnn_b):
    r2 = lambda v: v.reshape(1, D)
    zeros_nd = jnp.zeros((NPT_LAST, D), jnp.float32)

    uv = _uv_precompute(x, We1)
    sh = senders + N
    idxA = jnp.concatenate([receivers[:EH], sh[:EH]])
    idxB = jnp.concatenate([receivers[EH:], sh[EH:]])

    gsA = _sc_gather_add_h(uv, idxA)
    gsB = _sc_gather_add_h(uv, idxB)

    def half_edges(gs_h, off, eout_prev=None):
        return _edge_mlp(edge_attr, gs_h, We1, r2(be1), We2, r2(be2),
                         We3, r2(be3), r2(lne_g), r2(lne_b),
                         ne=EH, block=4000, ea_off=off, eout_prev=eout_prev)

    msgA, eoutA = half_edges(gsA, 0)
    msgB, edge_out = half_edges(gsB, EH // 4000, eout_prev=eoutA)

    sidxA = jnp.concatenate([receivers[:EH], senders[:EH]])
    sidxB = jnp.concatenate([receivers[EH:], senders[EH:]])
    aggA = _sc_scatter_h(msgA, sidxA, zeros_nd)
    aggB = _sc_scatter_h(msgB, sidxB, zeros_nd)

    x_out = _node_mlp(x, aggA, aggB, Wn1, r2(bn1), Wn2, r2(bn2), Wn3, r2(bn3),
                      r2(lnn_g), r2(lnn_b), block=1000)

    return x_out, edge_out


# speedup vs baseline: 1.0097x; 1.0097x over previous
import functools

import jax
import jax.numpy as jnp
from jax import lax
from jax.experimental import pallas as pl
from jax.experimental.pallas import tpu as pltpu
from jax.experimental.pallas import tpu_sc as plsc

N = 10000
E = 320000
D = 128

NC = 2
NS = 16
NW = NC * NS

CH = 128

NPT = 632
NPT_LAST = N - 15 * NPT

_sc_mesh = plsc.VectorSubcoreMesh(core_axis_name="c", subcore_axis_name="s")


def _make_sc_gather_add(ne):
    orows = ne // NW
    cf = orows // CH
    ct = orows - cf * CH
    assert ne % NW == 0 and ct % 8 == 0 and ct > 0 and cf % 2 == 1

    @functools.partial(
        pl.kernel,
        out_type=jax.ShapeDtypeStruct((ne, D), jnp.float32),
        mesh=_sc_mesh,
        scratch_types=[
            pltpu.VMEM((CH,), jnp.int32),
            pltpu.VMEM((CH,), jnp.int32),
            pltpu.VMEM((CH,), jnp.int32),
            pltpu.VMEM((CH,), jnp.int32),
            pltpu.VMEM((ct,), jnp.int32),
            pltpu.VMEM((ct,), jnp.int32),
            pltpu.VMEM((CH, D), jnp.float32),
            pltpu.VMEM((CH, D), jnp.float32),
            pltpu.VMEM((CH, D), jnp.float32),
            pltpu.VMEM((CH, D), jnp.float32),
            pltpu.VMEM((ct, D), jnp.float32),
            pltpu.VMEM((ct, D), jnp.float32),
            pltpu.SemaphoreType.DMA,
            pltpu.SemaphoreType.DMA,
            pltpu.SemaphoreType.DMA,
            pltpu.SemaphoreType.DMA,
            pltpu.SemaphoreType.DMA,
            pltpu.SemaphoreType.DMA,
            pltpu.SemaphoreType.DMA,
            pltpu.SemaphoreType.DMA,
            pltpu.SemaphoreType.DMA,
            pltpu.SemaphoreType.DMA,
        ],
    )
    def sc_gather_add(uv_hbm, idx_hbm, out_hbm,
                      ir0, ir1, is0, is1, irt, ist,
                      ra0, ra1, rb0, rb1, rat, rbt,
                      sir0, sir1, sis0, sis1, sga0, sga1, sgb0, sgb1, ss0, ss1):
        c = lax.axis_index("c")
        s = lax.axis_index("s")
        base_w = (s * NC + c) * orows
        ir, isv = (ir0, ir1), (is0, is1)
        ra, rb = (ra0, ra1), (rb0, rb1)
        sir, sis = (sir0, sir1), (sis0, sis1)
        sga, sgb = (sga0, sga1), (sgb0, sgb1)
        ss = (ss0, ss1)

        def start_idx(j, b):
            pltpu.async_copy(idx_hbm.at[pl.ds(base_w + j * CH, CH)], ir[b], sir[b])
            pltpu.async_copy(idx_hbm.at[pl.ds(ne + base_w + j * CH, CH)],
                             isv[b], sis[b])

        def wait_idx(b):
            pltpu.make_async_copy(idx_hbm.at[pl.ds(base_w, CH)], ir[b], sir[b]).wait()
            pltpu.make_async_copy(idx_hbm.at[pl.ds(base_w, CH)], isv[b], sis[b]).wait()

        def start_gathers(b):
            pltpu.async_copy(uv_hbm.at[ir[b]], ra[b], sga[b])
            pltpu.async_copy(uv_hbm.at[isv[b]], rb[b], sgb[b])

        def wait_gathers(b):
            pltpu.make_async_copy(uv_hbm.at[ir[b]], ra[b], sga[b]).wait()
            pltpu.make_async_copy(uv_hbm.at[isv[b]], rb[b], sgb[b]).wait()

        def vadd(dst, src, nrow):
            @pl.loop(0, nrow)
            def _(r):
                for q in range(D // 16):
                    sl = pl.ds(q * 16, 16)
                    dst[r, sl] = dst[r, sl] + src[r, sl]

        def start_store(j, b):
            pltpu.async_copy(ra[b], out_hbm.at[pl.ds(base_w + j * CH, CH)], ss[b])

        def wait_store(b):
            pltpu.make_async_copy(ra[b], out_hbm.at[pl.ds(base_w, CH)], ss[b]).wait()

        def chunk(j, b, wait_prev_store, start_next):
            nb = 1 - b
            if start_next:
                start_idx(j + 1, nb)
            wait_gathers(b)
            if start_next:
                wait_idx(nb)
                if wait_prev_store:
                    wait_store(nb)
                start_gathers(nb)
            vadd(ra[b], rb[b], CH)
            start_store(j, b)

        pltpu.sync_copy(idx_hbm.at[pl.ds(base_w, CH)], ir0)
        pltpu.sync_copy(idx_hbm.at[pl.ds(ne + base_w, CH)], is0)
        start_gathers(0)
        chunk(0, 0, wait_prev_store=False, start_next=True)

        @pl.loop(1, cf - 2, step=2)
        def _(j0):
            chunk(j0, 1, wait_prev_store=True, start_next=True)
            chunk(j0 + 1, 0, wait_prev_store=True, start_next=True)

        chunk(cf - 2, 1, wait_prev_store=True, start_next=True)
        chunk(cf - 1, 0, wait_prev_store=False, start_next=False)

        tb = base_w + cf * CH
        pltpu.sync_copy(idx_hbm.at[pl.ds(tb, ct)], irt)
        pltpu.sync_copy(idx_hbm.at[pl.ds(ne + tb, ct)], ist)
        pltpu.async_copy(uv_hbm.at[irt], rat, sga0).wait()
        pltpu.async_copy(uv_hbm.at[ist], rbt, sgb0).wait()
        vadd(rat, rbt, ct)
        pltpu.sync_copy(rat, out_hbm.at[pl.ds(tb, ct)])

        wait_store(1)
        wait_store(0)

    return sc_gather_add


def _make_sc_scatter(ne):
    srows = ne // NS
    sfull = srows // CH
    stail = srows - sfull * CH
    assert ne % NS == 0 and stail % 8 == 0 and stail > 0 and sfull % 2 == 0

    @functools.partial(
        pl.kernel,
        out_type=jax.ShapeDtypeStruct((2 * N, D), jnp.float32),
        mesh=_sc_mesh,
        scratch_types=[
            pltpu.VMEM((CH,), jnp.int32),
            pltpu.VMEM((CH,), jnp.int32),
            pltpu.VMEM((stail,), jnp.int32),
            pltpu.VMEM((CH, D), jnp.float32),
            pltpu.VMEM((CH, D), jnp.float32),
            pltpu.VMEM((stail, D), jnp.float32),
            pltpu.VMEM_SHARED((N, D), jnp.float32),
            pltpu.SemaphoreType.DMA,
            pltpu.SemaphoreType.DMA,
            pltpu.SemaphoreType.DMA,
            pltpu.SemaphoreType.DMA,
            pltpu.SemaphoreType.DMA,
            pltpu.SemaphoreType.DMA,
        ],
    )
    def sc_scatter(msg_hbm, idx_hbm, zero_hbm, out_hbm,
                   idx0, idx1, idxt, r0, r1, rt, acc,
                   si0, si1, sm0, sm1, sa0, sa1):
        c = lax.axis_index("c")
        s = lax.axis_index("s")
        idxb, rows = (idx0, idx1), (r0, r1)
        si, sm, sa = (si0, si1), (sm0, sm1), (sa0, sa1)
        ebase = s * srows

        def start_loads(j, b):
            pltpu.async_copy(idx_hbm.at[pl.ds(c * ne + ebase + j * CH, CH)],
                             idxb[b], si[b])
            pltpu.async_copy(msg_hbm.at[pl.ds(ebase + j * CH, CH)], rows[b], sm[b])

        def wait_loads(b):
            pltpu.make_async_copy(idx_hbm.at[pl.ds(ebase, CH)], idxb[b], si[b]).wait()
            pltpu.make_async_copy(msg_hbm.at[pl.ds(ebase, CH)], rows[b], sm[b]).wait()

        def start_scatter(b):
            pltpu.async_copy(rows[b], acc.at[idxb[b]], sa[b], add=True)

        def wait_scatter(b):
            pltpu.make_async_copy(rows[b], acc.at[idxb[b]], sa[b]).wait()

        start_loads(0, 0)

        abase = s * NPT
        pltpu.sync_copy(zero_hbm.at[pl.ds(0, NPT_LAST)],
                        acc.at[pl.ds(abase, NPT_LAST)])

        @pl.when(s < NS - 1)
        def _():
            pltpu.sync_copy(zero_hbm.at[pl.ds(0, NPT - NPT_LAST)],
                            acc.at[pl.ds(abase + NPT_LAST, NPT - NPT_LAST)])

        plsc.subcore_barrier()

        def chunk(j, b, wait_prev_scatter, start_next):
            nb = 1 - b
            if wait_prev_scatter:
                wait_scatter(nb)
            if start_next:
                start_loads(j + 1, nb)
            wait_loads(b)
            start_scatter(b)

        chunk(0, 0, wait_prev_scatter=False, start_next=True)

        @pl.loop(1, sfull - 1, step=2)
        def _(j0):
            chunk(j0, 1, wait_prev_scatter=True, start_next=True)
            chunk(j0 + 1, 0, wait_prev_scatter=True, start_next=True)

        chunk(sfull - 1, 1, wait_prev_scatter=True, start_next=False)
        wait_scatter(1)

        tb = ebase + sfull * CH
        pltpu.sync_copy(idx_hbm.at[pl.ds(c * ne + tb, stail)], idxt)
        pltpu.sync_copy(msg_hbm.at[pl.ds(tb, stail)], rt)
        pltpu.sync_copy(rt, acc.at[idxt], add=True)
        plsc.subcore_barrier()

        pltpu.sync_copy(acc.at[pl.ds(abase, NPT_LAST)],
                        out_hbm.at[pl.ds(c * N + abase, NPT_LAST)])

        @pl.when(s < NS - 1)
        def _():
            pltpu.sync_copy(acc.at[pl.ds(abase + NPT_LAST, NPT - NPT_LAST)],
                            out_hbm.at[pl.ds(c * N + abase + NPT_LAST,
                                             NPT - NPT_LAST)])

    return sc_scatter


def _uv_body(x_ref, w_ref, out_ref):
    out_ref[...] = jnp.dot(x_ref[...], w_ref[...],
                           preferred_element_type=jnp.float32)


def _uv_precompute(x, We1, block=1000):
    nb = N // block
    return pl.pallas_call(
        _uv_body,
        grid=(2, nb),
        in_specs=[
            pl.BlockSpec((block, D), lambda k, i: (i, 0)),
            pl.BlockSpec((D, D), lambda k, i: (k + 1, 0)),
        ],
        out_specs=pl.BlockSpec((block, D), lambda k, i: (k * nb + i, 0)),
        out_shape=jax.ShapeDtypeStruct((2 * N, D), jnp.float32),
    )(x, We1)


def _edge_body(ea_ref, gs_ref, w1_ref, b1_ref, w2_ref, b2_ref,
               w3_ref, b3_ref, g_ref, b_ref, msg_ref, eout_ref):
    ea = ea_ref[...]
    f32 = jnp.float32
    h = jnp.dot(ea, w1_ref[0:D, :], preferred_element_type=f32)
    h += gs_ref[...]
    h = jnp.tanh(h + b1_ref[...])
    h = jnp.tanh(jnp.dot(h, w2_ref[...], preferred_element_type=f32) + b2_ref[...])
    h = jnp.dot(h, w3_ref[...], preferred_element_type=f32) + b3_ref[...]
    mu = jnp.mean(h, axis=-1, keepdims=True)
    hc = h - mu
    var = jnp.mean(hc * hc, axis=-1, keepdims=True)
    m = hc * lax.rsqrt(var + 1e-5) * g_ref[...] + b_ref[...]
    msg_ref[...] = m
    eout_ref[...] = ea + m


def _edge_mlp(edge_attr, gs, We1, be1, We2, be2, We3, be3, g, b, ne, block,
              ea_off, eout_prev=None):
    nb = ne // block
    row = lambda i: (i, 0)
    eout_row = lambda i: (i + ea_off, 0)
    full = lambda shape: pl.BlockSpec(shape, lambda i: (0, 0))
    in_specs = [
        pl.BlockSpec((block, D), eout_row),
        pl.BlockSpec((block, D), row),
        full((3 * D, D)), full((1, D)),
        full((D, D)), full((1, D)),
        full((D, D)), full((1, D)),
        full((1, D)), full((1, D)),
    ]
    args = [edge_attr, gs, We1, be1, We2, be2, We3, be3, g, b]
    aliases = {}
    body = _edge_body
    if eout_prev is not None:
        in_specs.append(pl.BlockSpec((8, D), lambda i: (0, 0)))
        args.append(eout_prev)
        aliases = {len(args) - 1: 1}
        body = lambda *refs: _edge_body(*refs[:10], *refs[11:])
    return pl.pallas_call(
        body,
        grid=(nb,),
        in_specs=in_specs,
        out_specs=[pl.BlockSpec((block, D), row),
                   pl.BlockSpec((block, D), eout_row)],
        out_shape=[
            jax.ShapeDtypeStruct((ne, D), jnp.float32),
            jax.ShapeDtypeStruct((E, D), jnp.float32),
        ],
        input_output_aliases=aliases,
    )(*args)


def _node_body(x_ref, ar_ref, as_ref, br_ref, bs_ref, w1_ref, b1_ref,
               w2_ref, b2_ref, w3_ref, b3_ref, g_ref, b_ref, out_ref):
    x = x_ref[...]
    agg = (ar_ref[...] + br_ref[...]) - (as_ref[...] + bs_ref[...])
    f32 = jnp.float32
    h = jnp.dot(x, w1_ref[0:D, :], preferred_element_type=f32)
    h += jnp.dot(agg, w1_ref[D:2 * D, :], preferred_element_type=f32)
    h = jnp.tanh(h + b1_ref[...])
    h = jnp.tanh(jnp.dot(h, w2_ref[...], preferred_element_type=f32) + b2_ref[...])
    h = jnp.dot(h, w3_ref[...], preferred_element_type=f32) + b3_ref[...]
    mu = jnp.mean(h, axis=-1, keepdims=True)
    hc = h - mu
    var = jnp.mean(hc * hc, axis=-1, keepdims=True)
    out_ref[...] = hc * lax.rsqrt(var + 1e-5) * g_ref[...] + b_ref[...] + x


def _node_mlp(x, aggA, aggB, Wn1, bn1, Wn2, bn2, Wn3, bn3, g, b, block):
    nb = N // block
    row = lambda i: (i, 0)
    shift = lambda i: (i + nb, 0)
    full = lambda shape: pl.BlockSpec(shape, lambda i: (0, 0))
    return pl.pallas_call(
        _node_body,
        grid=(nb,),
        in_specs=[
            pl.BlockSpec((block, D), row),
            pl.BlockSpec((block, D), row),
            pl.BlockSpec((block, D), shift),
            pl.BlockSpec((block, D), row),
            pl.BlockSpec((block, D), shift),
            full((2 * D, D)), full((1, D)),
            full((D, D)), full((1, D)),
            full((D, D)), full((1, D)),
            full((1, D)), full((1, D)),
        ],
        out_specs=pl.BlockSpec((block, D), row),
        out_shape=jax.ShapeDtypeStruct((N, D), jnp.float32),
    )(x, aggA, aggA, aggB, aggB, Wn1, bn1, Wn2, bn2, Wn3, bn3, g, b)


EH = E // 2
_sc_gather_add_h = _make_sc_gather_add(EH)
_sc_scatter_h = _make_sc_scatter(EH)


@jax.jit
def kernel(x, senders, receivers, edge_attr,
           We1, be1, We2, be2, We3, be3, lne_g, lne_b,
           Wn1, bn1, Wn2, bn2, Wn3, bn3, lnn_g, lnn_b):
    r2 = lambda v: v.reshape(1, D)
    zeros_nd = jnp.zeros((NPT_LAST, D), jnp.float32)

    uv = _uv_precompute(x, We1)
    sh = senders + N
    idxA = jnp.concatenate([receivers[:EH], sh[:EH]])
    idxB = jnp.concatenate([receivers[EH:], sh[EH:]])

    gsA = _sc_gather_add_h(uv, idxA)
    gsB = _sc_gather_add_h(uv, idxB)

    def half_edges(gs_h, off, eout_prev=None):
        return _edge_mlp(edge_attr, gs_h, We1, r2(be1), We2, r2(be2),
                         We3, r2(be3), r2(lne_g), r2(lne_b),
                         ne=EH, block=3200, ea_off=off, eout_prev=eout_prev)

    msgA, eoutA = half_edges(gsA, 0)
    msgB, edge_out = half_edges(gsB, EH // 3200, eout_prev=eoutA)

    sidxA = jnp.concatenate([receivers[:EH], senders[:EH]])
    sidxB = jnp.concatenate([receivers[EH:], senders[EH:]])
    aggA = _sc_scatter_h(msgA, sidxA, zeros_nd)
    aggB = _sc_scatter_h(msgB, sidxB, zeros_nd)

    x_out = _node_mlp(x, aggA, aggB, Wn1, r2(bn1), Wn2, r2(bn2), Wn3, r2(bn3),
                      r2(lnn_g), r2(lnn_b), block=1000)

    return x_out, edge_out
